# baseline (speedup 1.0000x reference)
import functools

import jax
import jax.numpy as jnp
from jax import lax
from jax.experimental import pallas as pl
from jax.experimental.pallas import tpu as pltpu
from jax.experimental.pallas import tpu_sc as plsc

_L = 16
_NW = 32


def _lse_body(t_ref, o_ref):
    x = t_ref[...]
    m = jnp.max(x, axis=1)
    s = jnp.sum(jnp.exp(x - m[:, None]), axis=1)
    o_ref[...] = m + jnp.log(s)


def _fin_body(inv_n, p_ref, o_ref):
    o_ref[...] = (jnp.sum(p_ref[...]) * inv_n).reshape(1, 1)


def _sc_gather_loss(table, idx_flat, tgt_flat, lse_table, n, v, c):
    b_per_w = n // _NW
    ch = 40
    nbuf = 2
    n_rounds = b_per_w // (ch * nbuf)
    table_flat = table.reshape(v * c, 1)
    mesh = plsc.VectorSubcoreMesh(core_axis_name="c", subcore_axis_name="s")

    @functools.partial(
        pl.kernel,
        out_type=[
            jax.ShapeDtypeStruct((n, c), jnp.float32),
            jax.ShapeDtypeStruct((_NW, _L), jnp.float32),
        ],
        mesh=mesh,
        compiler_params=pltpu.CompilerParams(
            needs_layout_passes=False, use_tc_tiling_on_sc=False),
        scratch_types=[
            pltpu.VMEM((b_per_w,), jnp.int32),
            pltpu.VMEM((b_per_w,), jnp.int32),
            pltpu.VMEM((v,), jnp.float32),
            pltpu.VMEM((b_per_w // 80, 80), jnp.int32),
            pltpu.VMEM((b_per_w, 1), jnp.float32),
            pltpu.VMEM((nbuf, ch, c), jnp.float32),
            pltpu.VMEM((_L,), jnp.float32),
            pltpu.SemaphoreType.DMA,
            pltpu.SemaphoreType.DMA,
            pltpu.SemaphoreType.DMA,
            pltpu.SemaphoreType.DMA,
            pltpu.SemaphoreType.DMA,
        ],
    )
    def sc_kernel(table_hbm, idx_hbm, tgt_hbm, lse_hbm, tabf_hbm,
                  out_hbm, part_hbm,
                  idx_v, tgt_v, lse_v, off_v, picked_v, rows_v, acc_v,
                  psem, gsem0, gsem1, ssem0, ssem1):
        gsems = (gsem0, gsem1)
        ssems = (ssem0, ssem1)
        wid = lax.axis_index("s") * 2 + lax.axis_index("c")
        base = wid * b_per_w
        pltpu.sync_copy(idx_hbm.at[pl.ds(base, b_per_w)], idx_v)
        pltpu.sync_copy(tgt_hbm.at[pl.ds(base, b_per_w)], tgt_v)

        pk = 80

        def off_body(j, _):
            k = j // (pk // _L)
            r = j - k * (pk // _L)
            sl = pl.ds(j * _L, _L)
            off_v[k, pl.ds(r * _L, _L)] = idx_v[sl] * c + tgt_v[sl]
            return 0
        lax.fori_loop(0, b_per_w // _L, off_body, 0)

        def picked_fire(k, _):
            pltpu.async_copy(
                tabf_hbm.at[off_v.at[k]],
                picked_v.at[pl.ds(k * pk, pk)], psem)
            return 0
        lax.fori_loop(0, b_per_w // pk, picked_fire, 0)

        def start_gather(g, b):
            pltpu.async_copy(
                table_hbm.at[idx_v.at[pl.ds(g * ch, ch)]],
                rows_v.at[b], gsems[b])

        def wait_gather(g, b):
            pltpu.make_async_copy(
                table_hbm.at[idx_v.at[pl.ds(0, ch)]],
                rows_v.at[b], gsems[b]).wait()

        def start_scatter(g, b):
            pltpu.async_copy(
                rows_v.at[b], out_hbm.at[pl.ds(base + g * ch, ch)], ssems[b])

        def wait_scatter(b):
            pltpu.make_async_copy(
                rows_v.at[b], out_hbm.at[pl.ds(base, ch)], ssems[b]).wait()

        for b in range(nbuf):
            start_gather(b, b)

        def round_body(s, carry):
            g0 = s * nbuf
            for b in range(nbuf):
                wait_gather(g0 + b, b)
                start_scatter(g0 + b, b)
            for b in range(nbuf):
                wait_scatter(b)
                nxt = g0 + b + nbuf

                @pl.when(nxt < n_rounds * nbuf)
                def _():
                    start_gather(nxt, b)
            return carry

        lax.fori_loop(0, n_rounds, round_body, 0)

        pltpu.sync_copy(lse_hbm, lse_v)

        def picked_drain(k, _):
            pltpu.make_async_copy(
                tabf_hbm.at[off_v.at[0]],
                picked_v.at[pl.ds(0, pk)], psem).wait()
            return 0
        lax.fori_loop(0, b_per_w // pk, picked_drain, 0)

        def loss_body(j, a):
            sl = pl.ds(j * _L, _L)
            lse16 = plsc.load_gather(lse_v, [idx_v[sl]])
            r16 = lax.iota(jnp.int32, _L) + j * _L
            p16 = plsc.load_gather(picked_v, [r16, jnp.zeros((_L,), jnp.int32)])
            return a + (lse16 - p16)

        acc = lax.fori_loop(0, b_per_w // _L, loss_body,
                            jnp.zeros((_L,), jnp.float32))
        acc_v[...] = acc
        pltpu.sync_copy(acc_v, part_hbm.at[wid])

    return sc_kernel(table, idx_flat, tgt_flat, lse_table, table_flat)


def kernel(idx, targets, table):
    b, t = idx.shape
    v, c = table.shape
    n = b * t
    idx_flat = idx.reshape(-1)
    tgt_flat = targets.reshape(-1)

    lse_table = pl.pallas_call(
        _lse_body,
        out_shape=jax.ShapeDtypeStruct((v,), jnp.float32),
    )(table)

    logits_flat, partials = _sc_gather_loss(
        table, idx_flat, tgt_flat, lse_table, n, v, c)

    loss2d = pl.pallas_call(
        functools.partial(_fin_body, 1.0 / n),
        out_shape=jax.ShapeDtypeStruct((1, 1), jnp.float32),
    )(partials)

    return logits_flat, loss2d[0, 0]


# speedup vs baseline: 2.5095x; 2.5095x over previous
import functools

import jax
import jax.numpy as jnp
from jax import lax
from jax.experimental import pallas as pl
from jax.experimental.pallas import tpu as pltpu
from jax.experimental.pallas import tpu_sc as plsc

_L = 16
_NW = 32


def _lse_body(t_ref, o_ref):
    x = t_ref[...]
    m = jnp.max(x, axis=1)
    s = jnp.sum(jnp.exp(x - m[:, None]), axis=1)
    o_ref[...] = m + jnp.log(s)


def _fin_body(inv_n, p_ref, o_ref):
    o_ref[...] = (jnp.sum(p_ref[...]) * inv_n).reshape(1, 1)


def _sc_gather_loss(table, idx_flat, tgt_flat, lse_table, n, v, c):
    b_per_w = n // _NW
    ch = 32
    nbuf = 2
    n_rounds = b_per_w // (ch * nbuf)
    mesh = plsc.VectorSubcoreMesh(core_axis_name="c", subcore_axis_name="s")

    @functools.partial(
        pl.kernel,
        out_type=[
            jax.ShapeDtypeStruct((n, c), jnp.float32),
            jax.ShapeDtypeStruct((_NW, _L), jnp.float32),
        ],
        mesh=mesh,
        compiler_params=pltpu.CompilerParams(
            needs_layout_passes=False, use_tc_tiling_on_sc=False),
        scratch_types=[
            pltpu.VMEM((b_per_w,), jnp.int32),
            pltpu.VMEM((b_per_w,), jnp.int32),
            pltpu.VMEM((v,), jnp.float32),
            pltpu.VMEM((nbuf, ch, c), jnp.float32),
            pltpu.VMEM((_L,), jnp.float32),
            pltpu.SemaphoreType.DMA,
            pltpu.SemaphoreType.DMA,
            pltpu.SemaphoreType.DMA,
            pltpu.SemaphoreType.DMA,
        ],
    )
    def sc_kernel(table_hbm, idx_hbm, tgt_hbm, lse_hbm, out_hbm, part_hbm,
                  idx_v, tgt_v, lse_v, rows_v, acc_v,
                  gsem0, gsem1, ssem0, ssem1):
        gsems = (gsem0, gsem1)
        ssems = (ssem0, ssem1)
        wid = lax.axis_index("s") * 2 + lax.axis_index("c")
        base = wid * b_per_w
        pltpu.sync_copy(idx_hbm.at[pl.ds(base, b_per_w)], idx_v)
        pltpu.sync_copy(tgt_hbm.at[pl.ds(base, b_per_w)], tgt_v)
        pltpu.sync_copy(lse_hbm, lse_v)

        def start_gather(g, b):
            pltpu.async_copy(
                table_hbm.at[idx_v.at[pl.ds(g * ch, ch)]],
                rows_v.at[b], gsems[b])

        def wait_gather(b):
            pltpu.make_async_copy(
                table_hbm.at[idx_v.at[pl.ds(0, ch)]],
                rows_v.at[b], gsems[b]).wait()

        def start_scatter(g, b):
            pltpu.async_copy(
                rows_v.at[b], out_hbm.at[pl.ds(base + g * ch, ch)], ssems[b])

        def wait_scatter(b):
            pltpu.make_async_copy(
                rows_v.at[b], out_hbm.at[pl.ds(base, ch)], ssems[b]).wait()

        for b in range(nbuf):
            start_gather(b, b)

        def round_body(s, acc):
            g0 = s * nbuf
            for b in range(nbuf):
                wait_gather(b)
                start_scatter(g0 + b, b)
            for b in range(nbuf):
                def inner(j, a, b=b, g=g0 + b):
                    sl = pl.ds(g * ch + j * _L, _L)
                    lse16 = plsc.load_gather(lse_v, [idx_v[sl]])
                    r16 = lax.iota(jnp.int32, _L) + j * _L
                    p16 = plsc.load_gather(rows_v.at[b], [r16, tgt_v[sl]])
                    return a + (lse16 - p16)
                acc = lax.fori_loop(0, ch // _L, inner, acc)
            for b in range(nbuf):
                wait_scatter(b)
                nxt = g0 + b + nbuf

                @pl.when(nxt < n_rounds * nbuf)
                def _():
                    start_gather(nxt, b)
            return acc

        acc = lax.fori_loop(0, n_rounds, round_body,
                            jnp.zeros((_L,), jnp.float32))
        acc_v[...] = acc
        pltpu.sync_copy(acc_v, part_hbm.at[wid])

    return sc_kernel(table, idx_flat, tgt_flat, lse_table)


def kernel(idx, targets, table):
    b, t = idx.shape
    v, c = table.shape
    n = b * t
    idx_flat = idx.reshape(-1)
    tgt_flat = targets.reshape(-1)

    lse_table = pl.pallas_call(
        _lse_body,
        out_shape=jax.ShapeDtypeStruct((v,), jnp.float32),
    )(table)

    logits_flat, partials = _sc_gather_loss(
        table, idx_flat, tgt_flat, lse_table, n, v, c)

    loss2d = pl.pallas_call(
        functools.partial(_fin_body, 1.0 / n),
        out_shape=jax.ShapeDtypeStruct((1, 1), jnp.float32),
    )(partials)

    return logits_flat, loss2d[0, 0]


# speedup vs baseline: 3.9374x; 1.5690x over previous
import functools

import jax
import jax.numpy as jnp
from jax import lax
from jax.experimental import pallas as pl
from jax.experimental.pallas import tpu as pltpu
from jax.experimental.pallas import tpu_sc as plsc

_L = 16
_NW = 32
_CP = 1024


def _lse_pad_body(t_ref, lse_ref, pad_ref):
    x = t_ref[...]
    m = jnp.max(x, axis=1)
    s = jnp.sum(jnp.exp(x - m[:, None]), axis=1)
    lse_ref[...] = m + jnp.log(s)
    v = x.shape[0]
    pad_ref[...] = jnp.concatenate(
        [x, jnp.zeros((v, _CP - x.shape[1]), jnp.float32)], axis=1)


def _fin_body(inv_n, p_ref, o_ref):
    o_ref[...] = (jnp.sum(p_ref[...]) * inv_n).reshape(1, 1)


def _sc_gather(table_pad, idx_flat, n, c):
    b_per_w = n // _NW
    ch = 32
    nbuf = 2
    n_rounds = b_per_w // (ch * nbuf)
    mesh = plsc.VectorSubcoreMesh(core_axis_name="c", subcore_axis_name="s")

    @functools.partial(
        pl.kernel,
        out_type=jax.ShapeDtypeStruct((n, _CP), jnp.float32),
        mesh=mesh,
        compiler_params=pltpu.CompilerParams(
            needs_layout_passes=False, use_tc_tiling_on_sc=True),
        scratch_types=[
            pltpu.VMEM((b_per_w,), jnp.int32),
            pltpu.VMEM((nbuf, ch, _CP), jnp.float32),
            pltpu.SemaphoreType.DMA,
            pltpu.SemaphoreType.DMA,
            pltpu.SemaphoreType.DMA,
            pltpu.SemaphoreType.DMA,
        ],
    )
    def sc_kernel(tab_hbm, idx_hbm, out_hbm, idx_v, rows_v,
                  gsem0, gsem1, ssem0, ssem1):
        gsems = (gsem0, gsem1)
        ssems = (ssem0, ssem1)
        wid = lax.axis_index("s") * 2 + lax.axis_index("c")
        base = wid * b_per_w
        pltpu.sync_copy(idx_hbm.at[pl.ds(base, b_per_w)], idx_v)

        def start_gather(g, b):
            pltpu.async_copy(
                tab_hbm.at[idx_v.at[pl.ds(g * ch, ch)]],
                rows_v.at[b], gsems[b])

        def wait_gather(b):
            pltpu.make_async_copy(
                tab_hbm.at[idx_v.at[pl.ds(0, ch)]],
                rows_v.at[b], gsems[b]).wait()

        def start_scatter(g, b):
            pltpu.async_copy(
                rows_v.at[b],
                out_hbm.at[pl.ds(base + g * ch, ch)], ssems[b])

        def wait_scatter(b):
            pltpu.make_async_copy(
                rows_v.at[b],
                out_hbm.at[pl.ds(base, ch)], ssems[b]).wait()

        for b in range(nbuf):
            start_gather(b, b)

        def round_body(s, carry):
            g0 = s * nbuf
            for b in range(nbuf):
                wait_gather(b)
                start_scatter(g0 + b, b)
            for b in range(nbuf):
                wait_scatter(b)
                nxt = g0 + b + nbuf

                @pl.when(nxt < n_rounds * nbuf)
                def _():
                    start_gather(nxt, b)
            return carry

        lax.fori_loop(0, n_rounds, round_body, 0)

    return sc_kernel(table_pad, idx_flat)


def _sc_loss(table_pick, idx_flat, tgt_flat, lse_table, n, v, c):
    b_per_w = n // _NW
    pk = 80
    n_pk = b_per_w // pk
    lanes = table_pick.shape[1]
    tiles_per_row = _CP // lanes
    mesh = plsc.VectorSubcoreMesh(core_axis_name="c", subcore_axis_name="s")

    @functools.partial(
        pl.kernel,
        out_type=jax.ShapeDtypeStruct((_NW, _L), jnp.float32),
        mesh=mesh,
        compiler_params=pltpu.CompilerParams(
            needs_layout_passes=False, use_tc_tiling_on_sc=False),
        scratch_types=[
            pltpu.VMEM((b_per_w,), jnp.int32),
            pltpu.VMEM((b_per_w,), jnp.int32),
            pltpu.VMEM((v,), jnp.float32),
            pltpu.VMEM((n_pk, pk), jnp.int32),
            pltpu.VMEM((2, pk, lanes), jnp.float32),
            pltpu.VMEM((_L,), jnp.float32),
            pltpu.SemaphoreType.DMA,
            pltpu.SemaphoreType.DMA,
        ],
    )
    def sc_kernel(tabp_hbm, idx_hbm, tgt_hbm, lse_hbm, part_hbm,
                  idx_v, tgt_v, lse_v, off_v, land_v, acc_v, psem0, psem1):
        psems = (psem0, psem1)
        wid = lax.axis_index("s") * 2 + lax.axis_index("c")
        base = wid * b_per_w
        pltpu.sync_copy(idx_hbm.at[pl.ds(base, b_per_w)], idx_v)
        pltpu.sync_copy(tgt_hbm.at[pl.ds(base, b_per_w)], tgt_v)
        pltpu.sync_copy(lse_hbm, lse_v)

        def off_body(j, _):
            k = j // (pk // _L)
            r = j - k * (pk // _L)
            sl = pl.ds(j * _L, _L)
            off_v[k, pl.ds(r * _L, _L)] = (
                idx_v[sl] * tiles_per_row
                + lax.shift_right_logical(tgt_v[sl], 7))
            return 0
        lax.fori_loop(0, b_per_w // _L, off_body, 0)

        def fire(k, s):
            pltpu.async_copy(
                tabp_hbm.at[off_v.at[k]], land_v.at[s], psems[s])

        def drain(s):
            pltpu.make_async_copy(
                tabp_hbm.at[off_v.at[0]], land_v.at[s], psems[s]).wait()

        fire(0, 0)

        def chunk_body(k2, acc):
            for s in range(2):
                k = k2 * 2 + s

                @pl.when(k + 1 < n_pk)
                def _():
                    fire(k + 1, 1 - s)
                drain(s)

                def loss_body(j, a):
                    sl = pl.ds(k * pk + j * _L, _L)
                    lse16 = plsc.load_gather(lse_v, [idx_v[sl]])
                    r16 = lax.iota(jnp.int32, _L) + j * _L
                    lane16 = lax.bitwise_and(tgt_v[sl], 127)
                    p16 = plsc.load_gather(land_v.at[s], [r16, lane16])
                    return a + (lse16 - p16)

                acc = lax.fori_loop(0, pk // _L, loss_body, acc)
            return acc

        acc = lax.fori_loop(0, n_pk // 2, chunk_body,
                            jnp.zeros((_L,), jnp.float32))
        acc_v[...] = acc
        pltpu.sync_copy(acc_v, part_hbm.at[wid])

    return sc_kernel(table_pick, idx_flat, tgt_flat, lse_table)


def kernel(idx, targets, table):
    b, t = idx.shape
    v, c = table.shape
    n = b * t
    idx_flat = idx.reshape(-1)
    tgt_flat = targets.reshape(-1)

    lse_table, table_pad = pl.pallas_call(
        _lse_pad_body,
        out_shape=[
            jax.ShapeDtypeStruct((v,), jnp.float32),
            jax.ShapeDtypeStruct((v, _CP), jnp.float32),
        ],
    )(table)

    logits_pad = _sc_gather(table_pad, idx_flat, n, c)
    logits_flat = logits_pad[:, :c]
    table_pick = table_pad.reshape(v * (_CP // 128), 128)
    partials = _sc_loss(table_pick, idx_flat, tgt_flat, lse_table, n, v, c)

    loss2d = pl.pallas_call(
        functools.partial(_fin_body, 1.0 / n),
        out_shape=jax.ShapeDtypeStruct((1, 1), jnp.float32),
    )(partials)

    return logits_flat, loss2d[0, 0]
